# baseline (speedup 1.0000x reference)
import functools

import jax
import jax.numpy as jnp
from jax.experimental import pallas as pl
from jax.experimental.pallas import tpu as pltpu

NUM_HEADS = 8
NUM_CODES = 1024
HEAD_DIM = 64
BL = 512


def _pq_kernel(z_ref, cb_ref, ext_ref, zq_ref, idx_ref, loss_ref):
    zb = z_ref[0]
    cb = cb_ref[0]
    ext = ext_ref[0]
    sims = jnp.dot(cb, zb, preferred_element_type=jnp.float32)
    m = jnp.max(sims, axis=0, keepdims=True)
    onehot = (sims == m).astype(jnp.float32)
    zq_plus = jnp.dot(ext, onehot, preferred_element_type=jnp.float32)
    zq = zq_plus[:HEAD_DIM]
    zq_ref[0] = zq
    idx_ref[0, 0, 0] = zq_plus[HEAD_DIM].astype(jnp.int32)
    part = jnp.sum((zb - zq) ** 2)

    lb = pl.program_id(2)

    @pl.when(lb == 0)
    def _init():
        loss_ref[0, 0, 0] = jnp.zeros((128,), jnp.float32)

    loss_ref[0, 0, 0] = loss_ref[0, 0, 0] + part


@functools.partial(jax.jit, static_argnames=("interpret",))
def kernel(z, codebooks, interpret=False):
    b, d_model, l = z.shape
    h, c, d = codebooks.shape

    iota = jnp.broadcast_to(jnp.arange(c, dtype=jnp.float32), (h, 8, c))
    cb_ext = jnp.concatenate([jnp.transpose(codebooks, (0, 2, 1)), iota], axis=1)

    grid = (h, b, l // BL)
    zq, idx, loss_parts = pl.pallas_call(
        _pq_kernel,
        grid=grid,
        in_specs=[
            pl.BlockSpec((1, d, BL), lambda hh, bb, lb: (bb, hh, lb)),
            pl.BlockSpec((1, c, d), lambda hh, bb, lb: (hh, 0, 0)),
            pl.BlockSpec((1, d + 8, c), lambda hh, bb, lb: (hh, 0, 0)),
        ],
        out_specs=[
            pl.BlockSpec((1, d, BL), lambda hh, bb, lb: (bb, hh, lb)),
            pl.BlockSpec((1, 1, 1, BL), lambda hh, bb, lb: (bb, hh, 0, lb)),
            pl.BlockSpec((1, 1, 1, 128), lambda hh, bb, lb: (hh, bb, 0, 0)),
        ],
        out_shape=[
            jax.ShapeDtypeStruct((b, d_model, l), jnp.float32),
            jax.ShapeDtypeStruct((b, h, 1, l), jnp.int32),
            jax.ShapeDtypeStruct((h, b, 1, 128), jnp.float32),
        ],
        compiler_params=pltpu.CompilerParams(
            dimension_semantics=("parallel", "parallel", "arbitrary"),
        ),
        interpret=interpret,
    )(z, codebooks, cb_ext)

    n_total = h * b * l * d
    vq_loss = 1.25 * jnp.sum(loss_parts[:, :, 0, 0]) / n_total
    return (zq, vq_loss, idx.reshape(b, h, l))


# speedup vs baseline: 1.2795x; 1.2795x over previous
import functools

import jax
import jax.numpy as jnp
from jax.experimental import pallas as pl
from jax.experimental.pallas import tpu as pltpu

NUM_HEADS = 8
NUM_CODES = 1024
HEAD_DIM = 64
BL = 1024
HALF = 512


def _pq_kernel(z_ref, cb_ref, ext_ref, zq_ref, idx_ref, loss_ref):
    cb = cb_ref[0]
    ext = ext_ref[0]
    part = jnp.float32(0.0)
    for s in range(BL // HALF):
        sl = pl.ds(s * HALF, HALF)
        zb = z_ref[0, :, sl]
        sims = jnp.dot(cb, zb, preferred_element_type=jnp.float32)
        m1 = jnp.max(sims.reshape(8, NUM_CODES // 8, HALF), axis=0)
        m = jnp.max(m1, axis=0, keepdims=True)
        onehot = (sims == m).astype(jnp.float32)
        zq_plus = jnp.dot(ext, onehot, preferred_element_type=jnp.float32)
        zq = zq_plus[:HEAD_DIM]
        zq_ref[0, :, sl] = zq
        idx_ref[0, 0, 0, sl] = zq_plus[HEAD_DIM].astype(jnp.int32)
        part = part + jnp.sum((zb - zq) ** 2)
    loss_ref[0, 0, 0] = jnp.zeros((128,), jnp.float32) + part


@functools.partial(jax.jit, static_argnames=("interpret",))
def kernel(z, codebooks, interpret=False):
    b, d_model, l = z.shape
    h, c, d = codebooks.shape

    iota = jnp.broadcast_to(jnp.arange(c, dtype=jnp.float32), (h, 8, c))
    cb_ext = jnp.concatenate([jnp.transpose(codebooks, (0, 2, 1)), iota], axis=1)

    grid = (h, b)
    zq, idx, loss_parts = pl.pallas_call(
        _pq_kernel,
        grid=grid,
        in_specs=[
            pl.BlockSpec((1, d, BL), lambda hh, bb: (bb, hh, 0)),
            pl.BlockSpec((1, c, d), lambda hh, bb: (hh, 0, 0)),
            pl.BlockSpec((1, d + 8, c), lambda hh, bb: (hh, 0, 0)),
        ],
        out_specs=[
            pl.BlockSpec((1, d, BL), lambda hh, bb: (bb, hh, 0)),
            pl.BlockSpec((1, 1, 1, BL), lambda hh, bb: (bb, hh, 0, 0)),
            pl.BlockSpec((1, 1, 1, 128), lambda hh, bb: (hh, bb, 0, 0)),
        ],
        out_shape=[
            jax.ShapeDtypeStruct((b, d_model, l), jnp.float32),
            jax.ShapeDtypeStruct((b, h, 1, l), jnp.int32),
            jax.ShapeDtypeStruct((h, b, 1, 128), jnp.float32),
        ],
        compiler_params=pltpu.CompilerParams(
            dimension_semantics=("parallel", "parallel"),
        ),
        interpret=interpret,
    )(z, codebooks, cb_ext)

    n_total = h * b * l * d
    vq_loss = 1.25 * jnp.sum(loss_parts[:, :, 0, 0]) / n_total
    return (zq, vq_loss, idx.reshape(b, h, l))


# speedup vs baseline: 1.2875x; 1.0063x over previous
import functools

import jax
import jax.numpy as jnp
from jax.experimental import pallas as pl
from jax.experimental.pallas import tpu as pltpu

NUM_HEADS = 8
NUM_CODES = 1024
HEAD_DIM = 64
BL = 1024
HALF = 512


def _pq_kernel(z_ref, cb_ref, zq_ref, idx_ref, loss_ref, aug_ref):
    cb = cb_ref[0]
    bb = pl.program_id(1)

    @pl.when(bb == 0)
    def _build_aug():
        code = jax.lax.broadcasted_iota(jnp.int32, (8, NUM_CODES), 1)
        row = jax.lax.broadcasted_iota(jnp.int32, (8, NUM_CODES), 0)
        hi = (code // 8).astype(jnp.float32)
        lo = (code % 8).astype(jnp.float32)
        extra = jnp.where(row == 0, hi, jnp.where(row == 1, lo, 0.0))
        aug_ref[...] = jnp.concatenate(
            [cb.T, extra], axis=0).astype(jnp.bfloat16)

    aug = aug_ref[...]
    part = jnp.float32(0.0)
    for s in range(BL // HALF):
        sl = pl.ds(s * HALF, HALF)
        zb = z_ref[0, :, sl]
        sims = jnp.dot(cb, zb, preferred_element_type=jnp.float32)
        m1 = jnp.max(sims.reshape(8, NUM_CODES // 8, HALF), axis=0)
        m = jnp.max(m1, axis=0, keepdims=True)
        onehot = (sims == m).astype(jnp.bfloat16)
        zq_plus = jnp.dot(aug, onehot,
                          preferred_element_type=jnp.float32)
        zq = zq_plus[:HEAD_DIM]
        zq_ref[0, :, sl] = zq
        idx_ref[0, 0, 0, sl] = (
            8.0 * zq_plus[HEAD_DIM] + zq_plus[HEAD_DIM + 1]).astype(jnp.int32)
        part = part + jnp.sum((zb - zq) ** 2)
    loss_ref[0, 0, 0] = jnp.zeros((128,), jnp.float32) + part


@functools.partial(jax.jit, static_argnames=("interpret",))
def kernel(z, codebooks, interpret=False):
    b, d_model, l = z.shape
    h, c, d = codebooks.shape

    grid = (h, b)
    zq, idx, loss_parts = pl.pallas_call(
        _pq_kernel,
        grid=grid,
        in_specs=[
            pl.BlockSpec((1, d, BL), lambda hh, bb: (bb, hh, 0)),
            pl.BlockSpec((1, c, d), lambda hh, bb: (hh, 0, 0)),
        ],
        out_specs=[
            pl.BlockSpec((1, d, BL), lambda hh, bb: (bb, hh, 0)),
            pl.BlockSpec((1, 1, 1, BL), lambda hh, bb: (bb, hh, 0, 0)),
            pl.BlockSpec((1, 1, 1, 128), lambda hh, bb: (hh, bb, 0, 0)),
        ],
        out_shape=[
            jax.ShapeDtypeStruct((b, d_model, l), jnp.float32),
            jax.ShapeDtypeStruct((b, h, 1, l), jnp.int32),
            jax.ShapeDtypeStruct((h, b, 1, 128), jnp.float32),
        ],
        scratch_shapes=[pltpu.VMEM((d + 8, c), jnp.bfloat16)],
        compiler_params=pltpu.CompilerParams(
            dimension_semantics=("parallel", "arbitrary"),
        ),
        interpret=interpret,
    )(z, codebooks)

    n_total = h * b * l * d
    vq_loss = 1.25 * jnp.sum(loss_parts[:, :, 0, 0]) / n_total
    return (zq, vq_loss, idx.reshape(b, h, l))


# speedup vs baseline: 1.6006x; 1.2431x over previous
import functools

import jax
import jax.numpy as jnp
from jax.experimental import pallas as pl
from jax.experimental.pallas import tpu as pltpu

NUM_HEADS = 8
NUM_CODES = 1024
HEAD_DIM = 64
HALF = 512


def _pq_kernel(z_ref, cb_ref, zq_ref, idx_ref, loss_ref, aug_ref):
    cb = cb_ref[0]

    code = jax.lax.broadcasted_iota(jnp.int32, (8, NUM_CODES), 1)
    row = jax.lax.broadcasted_iota(jnp.int32, (8, NUM_CODES), 0)
    hi = (code // 8).astype(jnp.float32)
    lo = (code % 8).astype(jnp.float32)
    extra = jnp.where(row == 0, hi, jnp.where(row == 1, lo, 0.0))
    aug_ref[...] = jnp.concatenate([cb.T, extra], axis=0).astype(jnp.bfloat16)
    aug = aug_ref[...]

    nb = z_ref.shape[0]
    nl = z_ref.shape[2] // HALF
    part = jnp.float32(0.0)
    for b in range(nb):
        for s in range(nl):
            sl = pl.ds(s * HALF, HALF)
            zb = z_ref[b, :, sl]
            sims = jnp.dot(cb, zb, preferred_element_type=jnp.float32)
            m1 = jnp.max(sims.reshape(8, NUM_CODES // 8, HALF), axis=0)
            m = jnp.max(m1, axis=0, keepdims=True)
            onehot = (sims == m).astype(jnp.bfloat16)
            zq_plus = jnp.dot(aug, onehot,
                              preferred_element_type=jnp.float32)
            zq = zq_plus[:HEAD_DIM]
            zq_ref[b, :, sl] = zq
            idx_ref[b, 0, 0, sl] = (
                8.0 * zq_plus[HEAD_DIM] + zq_plus[HEAD_DIM + 1]).astype(jnp.int32)
            part = part + jnp.sum((zb - zq) ** 2)
    loss_ref[0, 0] = jnp.zeros((128,), jnp.float32) + part


@functools.partial(jax.jit, static_argnames=("interpret",))
def kernel(z, codebooks, interpret=False):
    b, d_model, l = z.shape
    h, c, d = codebooks.shape

    grid = (h,)
    zq, idx, loss_parts = pl.pallas_call(
        _pq_kernel,
        grid=grid,
        in_specs=[
            pl.BlockSpec((b, d, l), lambda hh: (0, hh, 0)),
            pl.BlockSpec((1, c, d), lambda hh: (hh, 0, 0)),
        ],
        out_specs=[
            pl.BlockSpec((b, d, l), lambda hh: (0, hh, 0)),
            pl.BlockSpec((b, 1, 1, l), lambda hh: (0, hh, 0, 0)),
            pl.BlockSpec((1, 1, 128), lambda hh: (hh, 0, 0)),
        ],
        out_shape=[
            jax.ShapeDtypeStruct((b, d_model, l), jnp.float32),
            jax.ShapeDtypeStruct((b, h, 1, l), jnp.int32),
            jax.ShapeDtypeStruct((h, 1, 128), jnp.float32),
        ],
        scratch_shapes=[pltpu.VMEM((d + 8, c), jnp.bfloat16)],
        compiler_params=pltpu.CompilerParams(
            dimension_semantics=("arbitrary",),
        ),
        interpret=interpret,
    )(z, codebooks)

    n_total = h * b * l * d
    vq_loss = 1.25 * jnp.sum(loss_parts[:, 0, 0]) / n_total
    return (zq, vq_loss, idx.reshape(b, h, l))


# speedup vs baseline: 1.6037x; 1.0019x over previous
import functools

import jax
import jax.numpy as jnp
from jax.experimental import pallas as pl
from jax.experimental.pallas import tpu as pltpu

NUM_HEADS = 8
NUM_CODES = 1024
HEAD_DIM = 64
HALF = 512


HEADS_PER_STEP = 2


def _pq_kernel(z_ref, cb_ref, zq_ref, idx_ref, loss_ref, aug_ref):
    code = jax.lax.broadcasted_iota(jnp.int32, (8, NUM_CODES), 1)
    row = jax.lax.broadcasted_iota(jnp.int32, (8, NUM_CODES), 0)
    hi = (code // 8).astype(jnp.float32)
    lo = (code % 8).astype(jnp.float32)
    extra = jnp.where(row == 0, hi, jnp.where(row == 1, lo, 0.0))
    for hh in range(HEADS_PER_STEP):
        aug_ref[hh] = jnp.concatenate(
            [cb_ref[hh].T, extra], axis=0).astype(jnp.bfloat16)

    nb = z_ref.shape[0]
    nl = z_ref.shape[2] // HALF
    part = jnp.float32(0.0)
    for hh in range(HEADS_PER_STEP):
        cb = cb_ref[hh]
        aug = aug_ref[hh]
        dsl = pl.ds(hh * HEAD_DIM, HEAD_DIM)
        for b in range(nb):
            for s in range(nl):
                sl = pl.ds(s * HALF, HALF)
                zb = z_ref[b, dsl, sl]
                sims = jnp.dot(cb, zb, preferred_element_type=jnp.float32)
                m1 = jnp.max(sims.reshape(8, NUM_CODES // 8, HALF), axis=0)
                m = jnp.max(m1, axis=0, keepdims=True)
                onehot = (sims == m).astype(jnp.bfloat16)
                zq_plus = jnp.dot(aug, onehot,
                                  preferred_element_type=jnp.float32)
                zq = zq_plus[:HEAD_DIM]
                zq_ref[b, dsl, sl] = zq
                idx_ref[b, hh, 0, sl] = (
                    8.0 * zq_plus[HEAD_DIM] + zq_plus[HEAD_DIM + 1]).astype(jnp.int32)
                part = part + jnp.sum((zb - zq) ** 2)
    loss_ref[0, 0] = jnp.zeros((128,), jnp.float32) + part


@functools.partial(jax.jit, static_argnames=("interpret",))
def kernel(z, codebooks, interpret=False):
    b, d_model, l = z.shape
    h, c, d = codebooks.shape

    hps = HEADS_PER_STEP
    grid = (h // hps,)
    zq, idx, loss_parts = pl.pallas_call(
        _pq_kernel,
        grid=grid,
        in_specs=[
            pl.BlockSpec((b, hps * d, l), lambda g: (0, g, 0)),
            pl.BlockSpec((hps, c, d), lambda g: (g, 0, 0)),
        ],
        out_specs=[
            pl.BlockSpec((b, hps * d, l), lambda g: (0, g, 0)),
            pl.BlockSpec((b, hps, 1, l), lambda g: (0, g, 0, 0)),
            pl.BlockSpec((1, 1, 128), lambda g: (g, 0, 0)),
        ],
        out_shape=[
            jax.ShapeDtypeStruct((b, d_model, l), jnp.float32),
            jax.ShapeDtypeStruct((b, h, 1, l), jnp.int32),
            jax.ShapeDtypeStruct((h // hps, 1, 128), jnp.float32),
        ],
        scratch_shapes=[pltpu.VMEM((hps, d + 8, c), jnp.bfloat16)],
        compiler_params=pltpu.CompilerParams(
            dimension_semantics=("arbitrary",),
        ),
        interpret=interpret,
    )(z, codebooks)

    n_total = h * b * l * d
    vq_loss = 1.25 * jnp.sum(loss_parts[:, 0, 0]) / n_total
    return (zq, vq_loss, idx.reshape(b, h, l))


# speedup vs baseline: 1.9630x; 1.2241x over previous
import functools

import jax
import jax.numpy as jnp
from jax.experimental import pallas as pl
from jax.experimental.pallas import tpu as pltpu

NUM_HEADS = 8
NUM_CODES = 1024
HEAD_DIM = 64
HALF = 1024


HEADS_PER_STEP = 2


def _pq_kernel(z_ref, cb_ref, zq_ref, idx_ref, loss_ref, aug_ref):
    code = jax.lax.broadcasted_iota(jnp.int32, (8, NUM_CODES), 1)
    row = jax.lax.broadcasted_iota(jnp.int32, (8, NUM_CODES), 0)
    hi = (code // 8).astype(jnp.float32)
    lo = (code % 8).astype(jnp.float32)
    extra = jnp.where(row == 0, hi, jnp.where(row == 1, lo, 0.0))
    for hh in range(HEADS_PER_STEP):
        aug_ref[hh] = jnp.concatenate(
            [cb_ref[hh].T, extra], axis=0).astype(jnp.bfloat16)

    nb = z_ref.shape[0]
    nl = z_ref.shape[2] // HALF
    part = jnp.float32(0.0)
    for hh in range(HEADS_PER_STEP):
        cb = cb_ref[hh]
        aug = aug_ref[hh]
        dsl = pl.ds(hh * HEAD_DIM, HEAD_DIM)
        for b in range(nb):
            for s in range(nl):
                sl = pl.ds(s * HALF, HALF)
                zb = z_ref[b, dsl, sl]
                sims = jnp.dot(cb, zb, preferred_element_type=jnp.float32)
                m1 = jnp.max(sims.reshape(8, NUM_CODES // 8, HALF), axis=0)
                m = jnp.max(m1, axis=0, keepdims=True)
                onehot = (sims == m).astype(jnp.bfloat16)
                zq_plus = jnp.dot(aug, onehot,
                                  preferred_element_type=jnp.float32)
                zq = zq_plus[:HEAD_DIM]
                zq_ref[b, dsl, sl] = zq
                idx_ref[b, hh, 0, sl] = (
                    8.0 * zq_plus[HEAD_DIM] + zq_plus[HEAD_DIM + 1]).astype(jnp.int32)
                part = part + jnp.sum((zb - zq) ** 2)
    loss_ref[0, 0] = jnp.zeros((128,), jnp.float32) + part


@functools.partial(jax.jit, static_argnames=("interpret",))
def kernel(z, codebooks, interpret=False):
    b, d_model, l = z.shape
    h, c, d = codebooks.shape

    hps = HEADS_PER_STEP
    grid = (h // hps,)
    zq, idx, loss_parts = pl.pallas_call(
        _pq_kernel,
        grid=grid,
        in_specs=[
            pl.BlockSpec((b, hps * d, l), lambda g: (0, g, 0)),
            pl.BlockSpec((hps, c, d), lambda g: (g, 0, 0)),
        ],
        out_specs=[
            pl.BlockSpec((b, hps * d, l), lambda g: (0, g, 0)),
            pl.BlockSpec((b, hps, 1, l), lambda g: (0, g, 0, 0)),
            pl.BlockSpec((1, 1, 128), lambda g: (g, 0, 0)),
        ],
        out_shape=[
            jax.ShapeDtypeStruct((b, d_model, l), jnp.float32),
            jax.ShapeDtypeStruct((b, h, 1, l), jnp.int32),
            jax.ShapeDtypeStruct((h // hps, 1, 128), jnp.float32),
        ],
        scratch_shapes=[pltpu.VMEM((hps, d + 8, c), jnp.bfloat16)],
        compiler_params=pltpu.CompilerParams(
            dimension_semantics=("arbitrary",),
        ),
        interpret=interpret,
    )(z, codebooks)

    n_total = h * b * l * d
    vq_loss = 1.25 * jnp.sum(loss_parts[:, 0, 0]) / n_total
    return (zq, vq_loss, idx.reshape(b, h, l))


# speedup vs baseline: 1.9695x; 1.0033x over previous
import functools

import jax
import jax.numpy as jnp
from jax.experimental import pallas as pl
from jax.experimental.pallas import tpu as pltpu

NUM_HEADS = 8
NUM_CODES = 1024
HEAD_DIM = 64
HALF = 1024


HEADS_PER_STEP = 2


def _pq_kernel(z_ref, cb_ref, zq_ref, idx_ref, loss_ref, aug_ref):
    code = jax.lax.broadcasted_iota(jnp.int32, (8, NUM_CODES), 1)
    row = jax.lax.broadcasted_iota(jnp.int32, (8, NUM_CODES), 0)
    hi = (code // 8).astype(jnp.float32)
    lo = (code % 8).astype(jnp.float32)
    extra = jnp.where(row == 0, hi, jnp.where(row == 1, lo, 0.0))
    for hh in range(HEADS_PER_STEP):
        aug_ref[hh] = jnp.concatenate([cb_ref[hh].T, extra], axis=0)

    nb = z_ref.shape[0]
    nl = z_ref.shape[2] // HALF
    part = jnp.float32(0.0)
    for hh in range(HEADS_PER_STEP):
        cb = cb_ref[hh]
        aug = aug_ref[hh]
        dsl = pl.ds(hh * HEAD_DIM, HEAD_DIM)
        for b in range(nb):
            for s in range(nl):
                sl = pl.ds(s * HALF, HALF)
                zb = z_ref[b, dsl, sl]
                sims = jnp.dot(cb, zb, preferred_element_type=jnp.float32)
                m1 = jnp.max(sims.reshape(8, NUM_CODES // 8, HALF), axis=0)
                m = jnp.max(m1, axis=0, keepdims=True)
                onehot = (sims == m).astype(jnp.float32)
                zq_plus = jnp.dot(aug, onehot,
                                  preferred_element_type=jnp.float32)
                zq = zq_plus[:HEAD_DIM]
                zq_ref[b, dsl, sl] = zq
                idx_ref[b, hh, 0, sl] = (
                    8.0 * zq_plus[HEAD_DIM] + zq_plus[HEAD_DIM + 1]).astype(jnp.int32)
                part = part + jnp.sum((zb - zq) ** 2)
    loss_ref[0, 0] = jnp.zeros((128,), jnp.float32) + part


@functools.partial(jax.jit, static_argnames=("interpret",))
def kernel(z, codebooks, interpret=False):
    b, d_model, l = z.shape
    h, c, d = codebooks.shape

    hps = HEADS_PER_STEP
    grid = (h // hps,)
    zq, idx, loss_parts = pl.pallas_call(
        _pq_kernel,
        grid=grid,
        in_specs=[
            pl.BlockSpec((b, hps * d, l), lambda g: (0, g, 0)),
            pl.BlockSpec((hps, c, d), lambda g: (g, 0, 0)),
        ],
        out_specs=[
            pl.BlockSpec((b, hps * d, l), lambda g: (0, g, 0)),
            pl.BlockSpec((b, hps, 1, l), lambda g: (0, g, 0, 0)),
            pl.BlockSpec((1, 1, 128), lambda g: (g, 0, 0)),
        ],
        out_shape=[
            jax.ShapeDtypeStruct((b, d_model, l), jnp.float32),
            jax.ShapeDtypeStruct((b, h, 1, l), jnp.int32),
            jax.ShapeDtypeStruct((h // hps, 1, 128), jnp.float32),
        ],
        scratch_shapes=[pltpu.VMEM((hps, d + 8, c), jnp.float32)],
        compiler_params=pltpu.CompilerParams(
            dimension_semantics=("parallel",),
        ),
        interpret=interpret,
    )(z, codebooks)

    n_total = h * b * l * d
    vq_loss = 1.25 * jnp.sum(loss_parts[:, 0, 0]) / n_total
    return (zq, vq_loss, idx.reshape(b, h, l))
